# Initial kernel scaffold; baseline (speedup 1.0000x reference)
import functools

import jax
import jax.numpy as jnp
from jax import lax
from jax.experimental import pallas as pl
from jax.experimental.pallas import tpu as pltpu
from jax.experimental.pallas import tpu_sc as plsc

D = 128
H = 512
NIMG = 16
MAXS = 15
NW = 32
NTILE = 16


def _round_up(x, m):
    return (x + m - 1) // m * m


def _mesh():
    return plsc.VectorSubcoreMesh(core_axis_name="c", subcore_axis_name="s")


@functools.cache
def _gather_call(N, M, C):
    R = M // NW
    nch = R // C

    @functools.partial(
        pl.kernel, mesh=_mesh(),
        out_type=jax.ShapeDtypeStruct((M, D), jnp.float32),
        scratch_types=[pltpu.VMEM((C,), jnp.int32),
                       pltpu.VMEM((C, D), jnp.float32),
                       pltpu.SemaphoreType.DMA])
    def k(table, idx, out, idx_v, rows_v, sem):
        wid = lax.axis_index("s") * 2 + lax.axis_index("c")
        base = wid * R

        def body(j, carry):
            e0 = base + j * C
            pltpu.sync_copy(idx.at[pl.ds(e0, C)], idx_v)
            pltpu.async_copy(table.at[idx_v], rows_v, sem).wait()
            pltpu.sync_copy(rows_v, out.at[pl.ds(e0, C)])
            return carry

        lax.fori_loop(0, nch, body, 0)

    return k


def _sc_gather(table, idx, chunk):
    return _gather_call(table.shape[0], idx.shape[0], chunk)(table, idx)


@functools.cache
def _counts_call(T_pad, O_pad):
    C = 128
    TPC = T_pad // NTILE
    nch = TPC // C
    CW = 16
    rpt = O_pad // NTILE

    @functools.partial(
        pl.kernel, mesh=_mesh(),
        out_type=(jax.ShapeDtypeStruct((O_pad, CW), jnp.float32),
                  jax.ShapeDtypeStruct((O_pad, CW), jnp.float32)),
        scratch_types=[pltpu.VMEM((C,), jnp.int32),
                       pltpu.VMEM((C, CW), jnp.float32),
                       pltpu.VMEM((C, CW), jnp.float32),
                       pltpu.VMEM_SHARED((O_pad, CW), jnp.float32),
                       pltpu.SemaphoreType.DMA])
    def k(sidx, oidx, cnt_s, cnt_o, idx_v, ones_v, zbuf, acc, sem):
        cid = lax.axis_index("c")
        sid = lax.axis_index("s")
        one16 = jnp.ones((CW,), jnp.float32)
        zero16 = jnp.zeros((CW,), jnp.float32)

        def fill(r, carry):
            ones_v[r, :] = one16
            zbuf[r, :] = zero16
            return carry

        lax.fori_loop(0, C, fill, 0)

        r0 = sid * rpt

        def zacc(j, carry):
            pltpu.sync_copy(zbuf, acc.at[pl.ds(r0 + j * C, C)])
            return carry

        lax.fori_loop(0, rpt // C, zacc, 0)
        plsc.subcore_barrier()

        def count_with(idxref):
            def body(j, carry):
                e0 = sid * TPC + j * C
                pltpu.sync_copy(idxref.at[pl.ds(e0, C)], idx_v)
                pltpu.sync_copy(ones_v, acc.at[idx_v], add=True)
                return carry
            lax.fori_loop(0, nch, body, 0)

        @pl.when(cid == 0)
        def _():
            count_with(sidx)

        @pl.when(cid == 1)
        def _():
            count_with(oidx)

        plsc.subcore_barrier()

        @pl.when(cid == 0)
        def _():
            pltpu.sync_copy(acc.at[pl.ds(r0, rpt)], cnt_s.at[pl.ds(r0, rpt)])

        @pl.when(cid == 1)
        def _():
            pltpu.sync_copy(acc.at[pl.ds(r0, rpt)], cnt_o.at[pl.ds(r0, rpt)])

    return k


def _sc_counts(s_pad, o_pad, O_pad):
    return _counts_call(s_pad.shape[0], O_pad)(s_pad, o_pad)


@functools.cache
def _scatter_call(T_pad, O_pad):
    C = 128
    TPC = T_pad // NTILE
    nch = TPC // C
    rpt = O_pad // NTILE

    @functools.partial(
        pl.kernel, mesh=_mesh(),
        out_type=jax.ShapeDtypeStruct((O_pad, 4 * D), jnp.float32),
        scratch_types=[pltpu.VMEM((C,), jnp.int32),
                       pltpu.VMEM((C, D), jnp.float32),
                       pltpu.VMEM((C, D), jnp.float32),
                       pltpu.VMEM_SHARED((O_pad, D), jnp.float32),
                       pltpu.SemaphoreType.DMA])
    def k(ns, no, sidx, oidx, out, idx_v, rows_v, zbuf, acc, sem):
        cid = lax.axis_index("c")
        sid = lax.axis_index("s")
        zero16 = jnp.zeros((16,), jnp.float32)

        def zb(r, carry):
            for c8 in range(D // 16):
                zbuf[r, pl.ds(c8 * 16, 16)] = zero16
            return carry

        lax.fori_loop(0, C, zb, 0)
        r0 = sid * rpt

        for cc_local in range(2):
            cc = cid * 2 + cc_local
            col0 = cc * D

            def zacc(j, carry):
                pltpu.sync_copy(zbuf, acc.at[pl.ds(r0 + j * C, C)])
                return carry

            lax.fori_loop(0, rpt // C, zacc, 0)
            plsc.subcore_barrier()

            def body(j, carry):
                e0 = sid * TPC + j * C
                pltpu.sync_copy(sidx.at[pl.ds(e0, C)], idx_v)
                pltpu.sync_copy(ns.at[pl.ds(e0, C), pl.ds(col0, D)], rows_v)
                pltpu.sync_copy(rows_v, acc.at[idx_v], add=True)
                pltpu.sync_copy(oidx.at[pl.ds(e0, C)], idx_v)
                pltpu.sync_copy(no.at[pl.ds(e0, C), pl.ds(col0, D)], rows_v)
                pltpu.sync_copy(rows_v, acc.at[idx_v], add=True)
                return carry

            lax.fori_loop(0, nch, body, 0)
            plsc.subcore_barrier()
            pltpu.sync_copy(acc.at[pl.ds(r0, rpt)],
                            out.at[pl.ds(r0, rpt), pl.ds(col0, D)])
            plsc.subcore_barrier()

    return k


def _sc_scatter(ns, no, s_pad, o_pad, O_pad):
    return _scatter_call(s_pad.shape[0], O_pad)(ns, no, s_pad, o_pad)


@functools.cache
def _graph_local_call(T_pad, O_pad):
    @functools.partial(
        pl.kernel, mesh=_mesh(),
        out_type=jax.ShapeDtypeStruct((NIMG, 16, 3, D), jnp.float32),
        scratch_types=[pltpu.VMEM((16,), jnp.int32),
                       pltpu.VMEM((16,), jnp.int32),
                       pltpu.VMEM((24,), jnp.int32),
                       pltpu.VMEM((24,), jnp.int32),
                       pltpu.VMEM((16, D), jnp.float32),
                       pltpu.VMEM((16, D), jnp.float32),
                       pltpu.VMEM((16, D), jnp.float32),
                       pltpu.VMEM((16,), jnp.int32),
                       pltpu.SemaphoreType.DMA])
    def k(aux_e, aux_c, sidx, oidx, ov, pv, out,
          ebuf, cbuf, swin, owin, bs, bpv, bo, gidx, sem):
        wid = lax.axis_index("s") * 2 + lax.axis_index("c")

        @pl.when(wid < NIMG)
        def _():
            b = wid
            pltpu.sync_copy(aux_e.at[b, pl.ds(0, 16)], ebuf)
            pltpu.sync_copy(aux_c.at[b, pl.ds(0, 16)], cbuf)
            evec = ebuf[...]
            cnt = jnp.max(cbuf[...])
            start = jnp.min(evec)
            lim = jnp.minimum(cnt, MAXS)
            lane = lax.broadcasted_iota(jnp.int32, (16,), 0)
            valid = lane < lim
            a0 = (start // 8) * 8
            off = start - a0
            pltpu.sync_copy(sidx.at[pl.ds(a0, 24)], swin)
            pltpu.sync_copy(oidx.at[pl.ds(a0, 24)], owin)
            sv = plsc.load_gather(swin, [lane + off])
            ovv = plsc.load_gather(owin, [lane + off])
            zrow = O_pad - 1
            sv = jnp.where(valid, sv, zrow)
            ovv = jnp.where(valid, ovv, zrow)
            pidx = jnp.where(valid, start + lane, T_pad - 1)
            gidx[...] = sv
            pltpu.async_copy(ov.at[gidx], bs, sem).wait()
            gidx[...] = pidx
            pltpu.async_copy(pv.at[gidx], bpv, sem).wait()
            gidx[...] = ovv
            pltpu.async_copy(ov.at[gidx], bo, sem).wait()
            pltpu.sync_copy(bs, out.at[b, :, 0])
            pltpu.sync_copy(bpv, out.at[b, :, 1])
            pltpu.sync_copy(bo, out.at[b, :, 2])

    return k


def _sc_graph_local(aux_e, aux_c, s_pad, o_pad, ov, pv):
    return _graph_local_call(s_pad.shape[0], ov.shape[0])(
        aux_e, aux_c, s_pad, o_pad, ov, pv)


def _edge_mlp(gs, pv, go, w1, b1, w2, b2, T, BE=1024):
    T_pad = gs.shape[0]
    grid = (T_pad // BE,)

    def kfn(gs_ref, pv_ref, go_ref, w1_ref, b1_ref, w2_ref, b2_ref,
            ns_ref, npv_ref, no_ref):
        i = pl.program_id(0)
        t = jnp.concatenate([gs_ref[...], pv_ref[...], go_ref[...]], axis=1)
        h = jnp.maximum(
            jnp.dot(t, w1_ref[...], preferred_element_type=jnp.float32)
            + b1_ref[...], 0.0)
        nt = jnp.maximum(
            jnp.dot(h, w2_ref[...], preferred_element_type=jnp.float32)
            + b2_ref[...], 0.0)
        rid = i * BE + lax.broadcasted_iota(jnp.int32, (BE, 1), 0)
        nt = jnp.where(rid < T, nt, 0.0)
        ns_ref[...] = nt[:, :H]
        npv_ref[...] = nt[:, H:H + D]
        no_ref[...] = nt[:, H + D:]

    return pl.pallas_call(
        kfn, grid=grid,
        in_specs=[pl.BlockSpec((BE, D), lambda i: (i, 0)),
                  pl.BlockSpec((BE, D), lambda i: (i, 0)),
                  pl.BlockSpec((BE, D), lambda i: (i, 0)),
                  pl.BlockSpec((3 * D, H), lambda i: (0, 0)),
                  pl.BlockSpec((1, H), lambda i: (0, 0)),
                  pl.BlockSpec((H, 2 * H + D), lambda i: (0, 0)),
                  pl.BlockSpec((1, 2 * H + D), lambda i: (0, 0))],
        out_specs=[pl.BlockSpec((BE, H), lambda i: (i, 0)),
                   pl.BlockSpec((BE, D), lambda i: (i, 0)),
                   pl.BlockSpec((BE, H), lambda i: (i, 0))],
        out_shape=[jax.ShapeDtypeStruct((T_pad, H), jnp.float32),
                   jax.ShapeDtypeStruct((T_pad, D), jnp.float32),
                   jax.ShapeDtypeStruct((T_pad, H), jnp.float32)],
    )(gs, pv, go, w1, b1, w2, b2)


def _node_mlp(pooled, cnt_s, cnt_o, w3, b3, w4, b4, O, BO=1024):
    O_pad = pooled.shape[0]
    grid = (O_pad // BO,)

    def kfn(pooled_ref, cs_ref, co_ref, w3_ref, b3_ref, w4_ref, b4_ref,
            nov_ref):
        i = pl.program_id(0)
        cnt = cs_ref[...][:, :1] + co_ref[...][:, :1]
        x = pooled_ref[...] * (1.0 / jnp.clip(cnt, 1.0, None))
        h2 = jnp.maximum(
            jnp.dot(x, w3_ref[...], preferred_element_type=jnp.float32)
            + b3_ref[...], 0.0)
        nov = jnp.maximum(
            jnp.dot(h2, w4_ref[...], preferred_element_type=jnp.float32)
            + b4_ref[...], 0.0)
        rid = i * BO + lax.broadcasted_iota(jnp.int32, (BO, 1), 0)
        nov_ref[...] = jnp.where(rid < O, nov, 0.0)

    return pl.pallas_call(
        kfn, grid=grid,
        in_specs=[pl.BlockSpec((BO, 4 * D), lambda i: (i, 0)),
                  pl.BlockSpec((BO, 16), lambda i: (i, 0)),
                  pl.BlockSpec((BO, 16), lambda i: (i, 0)),
                  pl.BlockSpec((H, H), lambda i: (0, 0)),
                  pl.BlockSpec((1, H), lambda i: (0, 0)),
                  pl.BlockSpec((H, D), lambda i: (0, 0)),
                  pl.BlockSpec((1, D), lambda i: (0, 0))],
        out_specs=pl.BlockSpec((BO, D), lambda i: (i, 0)),
        out_shape=jax.ShapeDtypeStruct((O_pad, D), jnp.float32),
    )(pooled, cnt_s, cnt_o, w3, b3, w4, b4)


def _segsum(x, ids3d, BS=1024):
    M = x.shape[0]
    NB = M // BS

    def kfn(x_ref, ids_ref, sum_ref, cnt_ref):
        i = pl.program_id(0)
        ids = ids_ref[0, 0, :]
        oh = (ids[None, :] == lax.broadcasted_iota(
            jnp.int32, (NIMG, BS), 0)).astype(jnp.float32)
        psum = jnp.dot(oh, x_ref[...], preferred_element_type=jnp.float32)
        pcnt = jnp.sum(oh, axis=1, keepdims=True)

        @pl.when(i == 0)
        def _():
            sum_ref[...] = jnp.zeros_like(sum_ref)
            cnt_ref[...] = jnp.zeros_like(cnt_ref)

        sum_ref[...] += psum
        cnt_ref[...] += jnp.broadcast_to(pcnt, (NIMG, D))

    return pl.pallas_call(
        kfn, grid=(NB,),
        in_specs=[pl.BlockSpec((BS, D), lambda i: (i, 0)),
                  pl.BlockSpec((1, 1, BS), lambda i: (i, 0, 0))],
        out_specs=[pl.BlockSpec((NIMG, D), lambda i: (0, 0)),
                   pl.BlockSpec((NIMG, D), lambda i: (0, 0))],
        out_shape=[jax.ShapeDtypeStruct((NIMG, D), jnp.float32),
                   jax.ShapeDtypeStruct((NIMG, D), jnp.float32)],
    )(x, ids3d)


def _final(osum, ocnt, psum, pcnt, wp, bp):
    def kfn(os_ref, oc_ref, ps_ref, pc_ref, wp_ref, bp_ref,
            gg_ref, auxe_ref, auxc_ref):
        ofea = os_ref[...] / jnp.clip(oc_ref[...], 1.0, None)
        pfea = ps_ref[...] / jnp.clip(pc_ref[...], 1.0, None)
        cat = jnp.concatenate([ofea, pfea], axis=1)
        gg_ref[...] = (jnp.dot(cat, wp_ref[...],
                               preferred_element_type=jnp.float32)
                       + bp_ref[...])
        counts = pc_ref[...][:, :1]
        tril = (lax.broadcasted_iota(jnp.int32, (NIMG, NIMG), 0)
                > lax.broadcasted_iota(jnp.int32, (NIMG, NIMG), 1)
                ).astype(jnp.float32)
        starts = jnp.dot(tril, counts, preferred_element_type=jnp.float32)
        lane = lax.broadcasted_iota(jnp.int32, (NIMG, D), 1)
        auxe_ref[...] = starts.astype(jnp.int32) + lane
        auxc_ref[...] = jnp.broadcast_to(counts.astype(jnp.int32), (NIMG, D))

    return pl.pallas_call(
        kfn, grid=(1,),
        in_specs=[pl.BlockSpec((NIMG, D), lambda i: (0, 0)),
                  pl.BlockSpec((NIMG, D), lambda i: (0, 0)),
                  pl.BlockSpec((NIMG, D), lambda i: (0, 0)),
                  pl.BlockSpec((NIMG, D), lambda i: (0, 0)),
                  pl.BlockSpec((2 * D, D), lambda i: (0, 0)),
                  pl.BlockSpec((1, D), lambda i: (0, 0))],
        out_specs=[pl.BlockSpec((NIMG, D), lambda i: (0, 0)),
                   pl.BlockSpec((NIMG, D), lambda i: (0, 0)),
                   pl.BlockSpec((NIMG, D), lambda i: (0, 0))],
        out_shape=[jax.ShapeDtypeStruct((NIMG, D), jnp.float32),
                   jax.ShapeDtypeStruct((NIMG, D), jnp.int32),
                   jax.ShapeDtypeStruct((NIMG, D), jnp.int32)],
    )(osum, ocnt, psum, pcnt, wp, bp)


def kernel(image, objs, boxes, triples, obj_to_img, triples_to_img, params):
    O = objs.shape[0]
    T = triples.shape[0]
    O_pad = _round_up(O, 2048)
    T_pad = _round_up(T + 24, 4096)
    padT = T_pad - T
    i32 = jnp.int32

    s = triples[:, 0].astype(i32)
    p = triples[:, 1].astype(i32)
    o = triples[:, 2].astype(i32)
    s_pad = jnp.concatenate([s, jnp.full((padT,), O, i32)])
    o_pad = jnp.concatenate([o, jnp.full((padT,), O, i32)])
    p_pad = jnp.concatenate([p, jnp.zeros((padT,), i32)])
    objs_pad = jnp.concatenate(
        [objs.astype(i32), jnp.zeros((O_pad - O,), i32)])
    oim3 = jnp.concatenate(
        [obj_to_img.astype(i32), jnp.full((O_pad - O,), NIMG, i32)]
    ).reshape(O_pad // 1024, 1, 1024)
    tim3 = jnp.concatenate(
        [triples_to_img.astype(i32), jnp.full((padT,), NIMG, i32)]
    ).reshape(T_pad // 1024, 1, 1024)

    cnt_s, cnt_o = _sc_counts(s_pad, o_pad, O_pad)

    ov = _sc_gather(params['obj_emb'], objs_pad, 64)
    pv = _sc_gather(params['pred_emb'], p_pad, 128)

    for gp in [params['gconv']] + list(params['gnet']):
        gs = _sc_gather(ov, s_pad, 128)
        go = _sc_gather(ov, o_pad, 128)
        ns, npv, no = _edge_mlp(
            gs, pv, go, gp['W1'], gp['b1'].reshape(1, H),
            gp['W2'], gp['b2'].reshape(1, 2 * H + D), T)
        pooled = _sc_scatter(ns, no, s_pad, o_pad, O_pad)
        ov = _node_mlp(pooled, cnt_s, cnt_o, gp['W3'],
                       gp['b3'].reshape(1, H), gp['W4'],
                       gp['b4'].reshape(1, D), O)
        pv = npv

    osum, ocnt = _segsum(ov, oim3)
    psum, pcnt = _segsum(pv, tim3)
    wp, bp = params['proj']
    gg, aux_e, aux_c = _final(osum, ocnt, psum, pcnt, wp, bp.reshape(1, D))
    glf = _sc_graph_local(aux_e, aux_c, s_pad, o_pad, ov, pv)
    graph_local = glf[:, :MAXS].reshape(NIMG, MAXS, 3 * D)
    return graph_local, gg


# speedup vs baseline: 1.3590x; 1.3590x over previous
import functools

import jax
import jax.numpy as jnp
from jax import lax
from jax.experimental import pallas as pl
from jax.experimental.pallas import tpu as pltpu
from jax.experimental.pallas import tpu_sc as plsc

D = 128
H = 512
NIMG = 16
MAXS = 15
NW = 32
NTILE = 16


def _round_up(x, m):
    return (x + m - 1) // m * m


def _mesh():
    return plsc.VectorSubcoreMesh(core_axis_name="c", subcore_axis_name="s")


@functools.cache
def _gather_call(N, M, C):
    R = M // NW
    nch = R // C

    @functools.partial(
        pl.kernel, mesh=_mesh(),
        out_type=jax.ShapeDtypeStruct((M, D), jnp.float32),
        scratch_types=[pltpu.VMEM((C,), jnp.int32),
                       pltpu.VMEM((C, D), jnp.float32),
                       pltpu.SemaphoreType.DMA])
    def k(table, idx, out, idx_v, rows_v, sem):
        wid = lax.axis_index("s") * 2 + lax.axis_index("c")
        base = wid * R

        def body(j, carry):
            e0 = base + j * C
            pltpu.sync_copy(idx.at[pl.ds(e0, C)], idx_v)
            pltpu.async_copy(table.at[idx_v], rows_v, sem).wait()
            pltpu.sync_copy(rows_v, out.at[pl.ds(e0, C)])
            return carry

        lax.fori_loop(0, nch, body, 0)

    return k


def _sc_gather(table, idx, chunk):
    return _gather_call(table.shape[0], idx.shape[0], chunk)(table, idx)


@functools.cache
def _counts_call(T_pad, O_pad):
    C = 128
    TPC = T_pad // NTILE
    nch = TPC // C
    CW = 128
    rpt = O_pad // NTILE

    @functools.partial(
        pl.kernel, mesh=_mesh(),
        out_type=(jax.ShapeDtypeStruct((O_pad, CW), jnp.float32),
                  jax.ShapeDtypeStruct((O_pad, CW), jnp.float32)),
        scratch_types=[pltpu.VMEM((C,), jnp.int32),
                       pltpu.VMEM((C, CW), jnp.float32),
                       pltpu.VMEM((C, CW), jnp.float32),
                       pltpu.VMEM_SHARED((O_pad, CW), jnp.float32),
                       pltpu.SemaphoreType.DMA])
    def k(sidx, oidx, cnt_s, cnt_o, idx_v, ones_v, zbuf, acc, sem):
        cid = lax.axis_index("c")
        sid = lax.axis_index("s")
        one16 = jnp.ones((CW,), jnp.float32)
        zero16 = jnp.zeros((CW,), jnp.float32)

        def fill(r, carry):
            ones_v[r, :] = one16
            zbuf[r, :] = zero16
            return carry

        lax.fori_loop(0, C, fill, 0)

        r0 = sid * rpt

        def zacc(j, carry):
            pltpu.sync_copy(zbuf, acc.at[pl.ds(r0 + j * C, C)])
            return carry

        lax.fori_loop(0, rpt // C, zacc, 0)
        plsc.subcore_barrier()

        def count_with(idxref):
            def body(j, carry):
                e0 = sid * TPC + j * C
                pltpu.sync_copy(idxref.at[pl.ds(e0, C)], idx_v)
                pltpu.sync_copy(ones_v, acc.at[idx_v], add=True)
                return carry
            lax.fori_loop(0, nch, body, 0)

        @pl.when(cid == 0)
        def _():
            count_with(sidx)

        @pl.when(cid == 1)
        def _():
            count_with(oidx)

        plsc.subcore_barrier()

        @pl.when(cid == 0)
        def _():
            pltpu.sync_copy(acc.at[pl.ds(r0, rpt)], cnt_s.at[pl.ds(r0, rpt)])

        @pl.when(cid == 1)
        def _():
            pltpu.sync_copy(acc.at[pl.ds(r0, rpt)], cnt_o.at[pl.ds(r0, rpt)])

    return k


def _sc_counts(s_pad, o_pad, O_pad):
    return _counts_call(s_pad.shape[0], O_pad)(s_pad, o_pad)


@functools.cache
def _scatter_call(T_pad, O_pad):
    C = 128
    TPC = T_pad // NTILE
    nch = TPC // C
    rpt = O_pad // NTILE

    @functools.partial(
        pl.kernel, mesh=_mesh(),
        out_type=jax.ShapeDtypeStruct((O_pad, 4 * D), jnp.float32),
        scratch_types=[pltpu.VMEM((C,), jnp.int32),
                       pltpu.VMEM((C, D), jnp.float32),
                       pltpu.VMEM((C, D), jnp.float32),
                       pltpu.VMEM_SHARED((O_pad, D), jnp.float32),
                       pltpu.SemaphoreType.DMA])
    def k(ns, no, sidx, oidx, out, idx_v, rows_v, zbuf, acc, sem):
        cid = lax.axis_index("c")
        sid = lax.axis_index("s")
        zero16 = jnp.zeros((16,), jnp.float32)

        def zb(r, carry):
            for c8 in range(D // 16):
                zbuf[r, pl.ds(c8 * 16, 16)] = zero16
            return carry

        lax.fori_loop(0, C, zb, 0)
        r0 = sid * rpt

        for cc_local in range(2):
            cc = cid * 2 + cc_local
            col0 = cc * D

            def zacc(j, carry):
                pltpu.sync_copy(zbuf, acc.at[pl.ds(r0 + j * C, C)])
                return carry

            lax.fori_loop(0, rpt // C, zacc, 0)
            plsc.subcore_barrier()

            def body(j, carry):
                e0 = sid * TPC + j * C
                pltpu.sync_copy(sidx.at[pl.ds(e0, C)], idx_v)
                pltpu.sync_copy(ns.at[pl.ds(e0, C), pl.ds(col0, D)], rows_v)
                pltpu.sync_copy(rows_v, acc.at[idx_v], add=True)
                pltpu.sync_copy(oidx.at[pl.ds(e0, C)], idx_v)
                pltpu.sync_copy(no.at[pl.ds(e0, C), pl.ds(col0, D)], rows_v)
                pltpu.sync_copy(rows_v, acc.at[idx_v], add=True)
                return carry

            lax.fori_loop(0, nch, body, 0)
            plsc.subcore_barrier()
            pltpu.sync_copy(acc.at[pl.ds(r0, rpt)],
                            out.at[pl.ds(r0, rpt), pl.ds(col0, D)])
            plsc.subcore_barrier()

    return k


def _sc_scatter(ns, no, s_pad, o_pad, O_pad):
    return _scatter_call(s_pad.shape[0], O_pad)(ns, no, s_pad, o_pad)


@functools.cache
def _graph_local_call(T_pad, O_pad):
    @functools.partial(
        pl.kernel, mesh=_mesh(),
        out_type=(jax.ShapeDtypeStruct((NIMG, 16, D), jnp.float32),
                  jax.ShapeDtypeStruct((NIMG, 16, D), jnp.float32),
                  jax.ShapeDtypeStruct((NIMG, 16, D), jnp.float32)),
        scratch_types=[pltpu.VMEM((16,), jnp.int32),
                       pltpu.VMEM((16,), jnp.int32),
                       pltpu.VMEM((32,), jnp.int32),
                       pltpu.VMEM((32,), jnp.int32),
                       pltpu.VMEM((16, D), jnp.float32),
                       pltpu.VMEM((16, D), jnp.float32),
                       pltpu.VMEM((16, D), jnp.float32),
                       pltpu.VMEM((16,), jnp.int32),
                       pltpu.SemaphoreType.DMA])
    def k(aux_e, aux_c, sidx, oidx, ov, pv, out_s, out_p, out_o,
          ebuf, cbuf, swin, owin, bs, bpv, bo, gidx, sem):
        wid = lax.axis_index("s") * 2 + lax.axis_index("c")
        b = wid & (NIMG - 1)
        pltpu.sync_copy(aux_e.at[b, pl.ds(0, 16)], ebuf)
        pltpu.sync_copy(aux_c.at[b, pl.ds(0, 16)], cbuf)
        evec = ebuf[...]
        cvec = cbuf[...]
        start = evec[0]
        lane = lax.broadcasted_iota(jnp.int32, (16,), 0)
        valid = lane < jnp.minimum(cvec, MAXS)
        a0 = pl.multiple_of((start >> 3) << 3, 8)
        off = start - a0
        pltpu.sync_copy(sidx.at[pl.ds(a0, 32)], swin)
        pltpu.sync_copy(oidx.at[pl.ds(a0, 32)], owin)
        svv = swin[pl.ds(off, 16)]
        ovvv = owin[pl.ds(off, 16)]
        zrow = O_pad - 1
        sv = jnp.where(valid, svv, zrow)
        ovv = jnp.where(valid, ovvv, zrow)
        pidx = jnp.where(valid, evec, T_pad - 1)
        gidx[...] = sv
        pltpu.async_copy(ov.at[gidx], bs, sem).wait()
        gidx[...] = pidx
        pltpu.async_copy(pv.at[gidx], bpv, sem).wait()
        gidx[...] = ovv
        pltpu.async_copy(ov.at[gidx], bo, sem).wait()
        pltpu.sync_copy(bs, out_s.at[b])
        pltpu.sync_copy(bpv, out_p.at[b])
        pltpu.sync_copy(bo, out_o.at[b])

    return k


def _sc_graph_local(aux_e, aux_c, s_pad, o_pad, ov, pv):
    return _graph_local_call(s_pad.shape[0], ov.shape[0])(
        aux_e, aux_c, s_pad, o_pad, ov, pv)


def _edge_mlp(gs, pv, go, w1, b1, w2, b2, T, BE=1024):
    T_pad = gs.shape[0]
    grid = (T_pad // BE,)

    def kfn(gs_ref, pv_ref, go_ref, w1_ref, b1_ref, w2_ref, b2_ref,
            ns_ref, npv_ref, no_ref):
        i = pl.program_id(0)
        t = jnp.concatenate([gs_ref[...], pv_ref[...], go_ref[...]], axis=1)
        h = jnp.maximum(
            jnp.dot(t, w1_ref[...], preferred_element_type=jnp.float32,
                    precision=jax.lax.Precision.HIGHEST)
            + b1_ref[...], 0.0)
        nt = jnp.maximum(
            jnp.dot(h, w2_ref[...], preferred_element_type=jnp.float32,
                    precision=jax.lax.Precision.HIGHEST)
            + b2_ref[...], 0.0)
        rid = i * BE + lax.broadcasted_iota(jnp.int32, (BE, 1), 0)
        nt = jnp.where(rid < T, nt, 0.0)
        ns_ref[...] = nt[:, :H]
        npv_ref[...] = nt[:, H:H + D]
        no_ref[...] = nt[:, H + D:]

    return pl.pallas_call(
        kfn, grid=grid,
        in_specs=[pl.BlockSpec((BE, D), lambda i: (i, 0)),
                  pl.BlockSpec((BE, D), lambda i: (i, 0)),
                  pl.BlockSpec((BE, D), lambda i: (i, 0)),
                  pl.BlockSpec((3 * D, H), lambda i: (0, 0)),
                  pl.BlockSpec((1, H), lambda i: (0, 0)),
                  pl.BlockSpec((H, 2 * H + D), lambda i: (0, 0)),
                  pl.BlockSpec((1, 2 * H + D), lambda i: (0, 0))],
        out_specs=[pl.BlockSpec((BE, H), lambda i: (i, 0)),
                   pl.BlockSpec((BE, D), lambda i: (i, 0)),
                   pl.BlockSpec((BE, H), lambda i: (i, 0))],
        out_shape=[jax.ShapeDtypeStruct((T_pad, H), jnp.float32),
                   jax.ShapeDtypeStruct((T_pad, D), jnp.float32),
                   jax.ShapeDtypeStruct((T_pad, H), jnp.float32)],
    )(gs, pv, go, w1, b1, w2, b2)


def _node_mlp(pooled, cnt_s, cnt_o, w3, b3, w4, b4, O, BO=1024):
    CW_CNT = cnt_s.shape[1]
    O_pad = pooled.shape[0]
    grid = (O_pad // BO,)

    def kfn(pooled_ref, cs_ref, co_ref, w3_ref, b3_ref, w4_ref, b4_ref,
            nov_ref):
        i = pl.program_id(0)
        cnt = cs_ref[...][:, :1] + co_ref[...][:, :1]
        x = pooled_ref[...] * (1.0 / jnp.clip(cnt, 1.0, None))
        h2 = jnp.maximum(
            jnp.dot(x, w3_ref[...], preferred_element_type=jnp.float32,
                    precision=jax.lax.Precision.HIGHEST)
            + b3_ref[...], 0.0)
        nov = jnp.maximum(
            jnp.dot(h2, w4_ref[...], preferred_element_type=jnp.float32,
                    precision=jax.lax.Precision.HIGHEST)
            + b4_ref[...], 0.0)
        rid = i * BO + lax.broadcasted_iota(jnp.int32, (BO, 1), 0)
        nov_ref[...] = jnp.where(rid < O, nov, 0.0)

    return pl.pallas_call(
        kfn, grid=grid,
        in_specs=[pl.BlockSpec((BO, 4 * D), lambda i: (i, 0)),
                  pl.BlockSpec((BO, CW_CNT), lambda i: (i, 0)),
                  pl.BlockSpec((BO, CW_CNT), lambda i: (i, 0)),
                  pl.BlockSpec((H, H), lambda i: (0, 0)),
                  pl.BlockSpec((1, H), lambda i: (0, 0)),
                  pl.BlockSpec((H, D), lambda i: (0, 0)),
                  pl.BlockSpec((1, D), lambda i: (0, 0))],
        out_specs=pl.BlockSpec((BO, D), lambda i: (i, 0)),
        out_shape=jax.ShapeDtypeStruct((O_pad, D), jnp.float32),
    )(pooled, cnt_s, cnt_o, w3, b3, w4, b4)


def _segsum(x, ids3d, BS=1024):
    M = x.shape[0]
    NB = M // BS

    def kfn(x_ref, ids_ref, sum_ref, cnt_ref):
        i = pl.program_id(0)
        ids = ids_ref[0, 0, :]
        oh = (ids[None, :] == lax.broadcasted_iota(
            jnp.int32, (NIMG, BS), 0)).astype(jnp.float32)
        psum = jnp.dot(oh, x_ref[...], preferred_element_type=jnp.float32,
                    precision=jax.lax.Precision.HIGHEST)
        pcnt = jnp.sum(oh, axis=1, keepdims=True)

        @pl.when(i == 0)
        def _():
            sum_ref[...] = jnp.zeros_like(sum_ref)
            cnt_ref[...] = jnp.zeros_like(cnt_ref)

        sum_ref[...] += psum
        cnt_ref[...] += jnp.broadcast_to(pcnt, (NIMG, D))

    return pl.pallas_call(
        kfn, grid=(NB,),
        in_specs=[pl.BlockSpec((BS, D), lambda i: (i, 0)),
                  pl.BlockSpec((1, 1, BS), lambda i: (i, 0, 0))],
        out_specs=[pl.BlockSpec((NIMG, D), lambda i: (0, 0)),
                   pl.BlockSpec((NIMG, D), lambda i: (0, 0))],
        out_shape=[jax.ShapeDtypeStruct((NIMG, D), jnp.float32),
                   jax.ShapeDtypeStruct((NIMG, D), jnp.float32)],
    )(x, ids3d)


def _final(osum, ocnt, psum, pcnt, wp, bp):
    def kfn(os_ref, oc_ref, ps_ref, pc_ref, wp_ref, bp_ref,
            gg_ref, auxe_ref, auxc_ref):
        ofea = os_ref[...] / jnp.clip(oc_ref[...], 1.0, None)
        pfea = ps_ref[...] / jnp.clip(pc_ref[...], 1.0, None)
        cat = jnp.concatenate([ofea, pfea], axis=1)
        gg_ref[...] = (jnp.dot(cat, wp_ref[...],
                               preferred_element_type=jnp.float32,
                    precision=jax.lax.Precision.HIGHEST)
                       + bp_ref[...])
        counts = pc_ref[...][:, :1]
        tril = (lax.broadcasted_iota(jnp.int32, (NIMG, NIMG), 0)
                > lax.broadcasted_iota(jnp.int32, (NIMG, NIMG), 1)
                ).astype(jnp.float32)
        starts = jnp.dot(tril, counts, preferred_element_type=jnp.float32,
                    precision=jax.lax.Precision.HIGHEST)
        lane = lax.broadcasted_iota(jnp.int32, (NIMG, D), 1)
        auxe_ref[...] = starts.astype(jnp.int32) + lane
        auxc_ref[...] = jnp.broadcast_to(counts.astype(jnp.int32), (NIMG, D))

    return pl.pallas_call(
        kfn, grid=(1,),
        in_specs=[pl.BlockSpec((NIMG, D), lambda i: (0, 0)),
                  pl.BlockSpec((NIMG, D), lambda i: (0, 0)),
                  pl.BlockSpec((NIMG, D), lambda i: (0, 0)),
                  pl.BlockSpec((NIMG, D), lambda i: (0, 0)),
                  pl.BlockSpec((2 * D, D), lambda i: (0, 0)),
                  pl.BlockSpec((1, D), lambda i: (0, 0))],
        out_specs=[pl.BlockSpec((NIMG, D), lambda i: (0, 0)),
                   pl.BlockSpec((NIMG, D), lambda i: (0, 0)),
                   pl.BlockSpec((NIMG, D), lambda i: (0, 0))],
        out_shape=[jax.ShapeDtypeStruct((NIMG, D), jnp.float32),
                   jax.ShapeDtypeStruct((NIMG, D), jnp.int32),
                   jax.ShapeDtypeStruct((NIMG, D), jnp.int32)],
    )(osum, ocnt, psum, pcnt, wp, bp)


def kernel(image, objs, boxes, triples, obj_to_img, triples_to_img, params):
    O = objs.shape[0]
    T = triples.shape[0]
    O_pad = _round_up(O, 2048)
    T_pad = _round_up(T + 24, 4096)
    padT = T_pad - T
    i32 = jnp.int32

    s = triples[:, 0].astype(i32)
    p = triples[:, 1].astype(i32)
    o = triples[:, 2].astype(i32)
    s_pad = jnp.concatenate([s, jnp.full((padT,), O, i32)])
    o_pad = jnp.concatenate([o, jnp.full((padT,), O, i32)])
    p_pad = jnp.concatenate([p, jnp.zeros((padT,), i32)])
    objs_pad = jnp.concatenate(
        [objs.astype(i32), jnp.zeros((O_pad - O,), i32)])
    oim3 = jnp.concatenate(
        [obj_to_img.astype(i32), jnp.full((O_pad - O,), NIMG, i32)]
    ).reshape(O_pad // 1024, 1, 1024)
    tim3 = jnp.concatenate(
        [triples_to_img.astype(i32), jnp.full((padT,), NIMG, i32)]
    ).reshape(T_pad // 1024, 1, 1024)

    cnt_s, cnt_o = _sc_counts(s_pad, o_pad, O_pad)

    ov = _sc_gather(params['obj_emb'], objs_pad, 64)
    pv = _sc_gather(params['pred_emb'], p_pad, 128)

    for gp in [params['gconv']] + list(params['gnet']):
        gs = _sc_gather(ov, s_pad, 128)
        go = _sc_gather(ov, o_pad, 128)
        ns, npv, no = _edge_mlp(
            gs, pv, go, gp['W1'], gp['b1'].reshape(1, H),
            gp['W2'], gp['b2'].reshape(1, 2 * H + D), T)
        pooled = _sc_scatter(ns, no, s_pad, o_pad, O_pad)
        ov = _node_mlp(pooled, cnt_s, cnt_o, gp['W3'],
                       gp['b3'].reshape(1, H), gp['W4'],
                       gp['b4'].reshape(1, D), O)
        pv = npv

    osum, ocnt = _segsum(ov, oim3)
    psum, pcnt = _segsum(pv, tim3)
    wp, bp = params['proj']
    gg, aux_e, aux_c = _final(osum, ocnt, psum, pcnt, wp, bp.reshape(1, D))
    gl_s, gl_p, gl_o = _sc_graph_local(aux_e, aux_c, s_pad, o_pad, ov, pv)
    glf = jnp.stack([gl_s, gl_p, gl_o], axis=2)
    graph_local = glf[:, :MAXS].reshape(NIMG, MAXS, 3 * D)
    return graph_local, gg


# speedup vs baseline: 2.3629x; 1.7388x over previous
import functools

import jax
import jax.numpy as jnp
from jax import lax
from jax.experimental import pallas as pl
from jax.experimental.pallas import tpu as pltpu
from jax.experimental.pallas import tpu_sc as plsc

D = 128
H = 512
NIMG = 16
MAXS = 15
NW = 32
NTILE = 16


def _round_up(x, m):
    return (x + m - 1) // m * m


def _mesh():
    return plsc.VectorSubcoreMesh(core_axis_name="c", subcore_axis_name="s")


@functools.cache
def _gather_call(N, M, C):
    R = M // NW
    nch = R // C
    assert nch % 2 == 0

    @functools.partial(
        pl.kernel, mesh=_mesh(),
        out_type=jax.ShapeDtypeStruct((M, D), jnp.float32),
        scratch_types=[pltpu.VMEM((C,), jnp.int32),
                       pltpu.VMEM((C,), jnp.int32),
                       pltpu.VMEM((C, D), jnp.float32),
                       pltpu.VMEM((C, D), jnp.float32),
                       pltpu.SemaphoreType.DMA,
                       pltpu.SemaphoreType.DMA,
                       pltpu.SemaphoreType.DMA,
                       pltpu.SemaphoreType.DMA,
                       pltpu.SemaphoreType.DMA,
                       pltpu.SemaphoreType.DMA])
    def k(table, idx, out, i0, i1, r0, r1, si0, si1, sg0, sg1, so0, so1):
        wid = lax.axis_index("s") * 2 + lax.axis_index("c")
        base = wid * R
        ivs, rvs = (i0, i1), (r0, r1)
        sis, sgs, sos = (si0, si1), (sg0, sg1), (so0, so1)

        def idx_cp(j, b):
            return pltpu.make_async_copy(
                idx.at[pl.ds(base + j * C, C)], ivs[b], sis[b])

        def out_cp(j, b):
            return pltpu.make_async_copy(
                rvs[b], out.at[pl.ds(base + j * C, C)], sos[b])

        idx_cp(0, 0).start()
        idx_cp(1, 1).start()

        def body(j2, carry):
            j0 = j2 * 2

            @pl.when(j2 > 0)
            def _():
                out_cp(j0 - 2, 0).wait()
                out_cp(j0 - 1, 1).wait()

            idx_cp(j0, 0).wait()
            g0 = pltpu.make_async_copy(table.at[ivs[0]], rvs[0], sgs[0])
            g0.start()
            idx_cp(j0 + 1, 1).wait()
            g1 = pltpu.make_async_copy(table.at[ivs[1]], rvs[1], sgs[1])
            g1.start()
            g0.wait()
            out_cp(j0, 0).start()
            g1.wait()
            out_cp(j0 + 1, 1).start()
            nj = j0 + 2
            njw = jnp.where(nj >= nch, 0, nj)
            idx_cp(njw, 0).start()
            idx_cp(njw + 1, 1).start()
            return carry

        lax.fori_loop(0, nch // 2, body, 0)
        out_cp(nch - 2, 0).wait()
        out_cp(nch - 1, 1).wait()
        idx_cp(0, 0).wait()
        idx_cp(1, 1).wait()

    return k


def _sc_gather(table, idx, chunk):
    return _gather_call(table.shape[0], idx.shape[0], chunk)(table, idx)


@functools.cache
def _counts_call(T_pad, O_pad):
    C = 128
    TPC = T_pad // NTILE
    nch = TPC // C
    CW = 128
    rpt = O_pad // NTILE

    @functools.partial(
        pl.kernel, mesh=_mesh(),
        out_type=(jax.ShapeDtypeStruct((O_pad, CW), jnp.float32),
                  jax.ShapeDtypeStruct((O_pad, CW), jnp.float32)),
        scratch_types=[pltpu.VMEM((C,), jnp.int32),
                       pltpu.VMEM((C, CW), jnp.float32),
                       pltpu.VMEM((C, CW), jnp.float32),
                       pltpu.VMEM_SHARED((O_pad, CW), jnp.float32),
                       pltpu.SemaphoreType.DMA])
    def k(sidx, oidx, cnt_s, cnt_o, idx_v, ones_v, zbuf, acc, sem):
        cid = lax.axis_index("c")
        sid = lax.axis_index("s")
        one16 = jnp.ones((CW,), jnp.float32)
        zero16 = jnp.zeros((CW,), jnp.float32)

        def fill(r, carry):
            ones_v[r, :] = one16
            zbuf[r, :] = zero16
            return carry

        lax.fori_loop(0, C, fill, 0)

        r0 = sid * rpt

        def zacc(j, carry):
            pltpu.sync_copy(zbuf, acc.at[pl.ds(r0 + j * C, C)])
            return carry

        lax.fori_loop(0, rpt // C, zacc, 0)
        plsc.subcore_barrier()

        def count_with(idxref):
            def body(j, carry):
                e0 = sid * TPC + j * C
                pltpu.sync_copy(idxref.at[pl.ds(e0, C)], idx_v)
                pltpu.sync_copy(ones_v, acc.at[idx_v], add=True)
                return carry
            lax.fori_loop(0, nch, body, 0)

        @pl.when(cid == 0)
        def _():
            count_with(sidx)

        @pl.when(cid == 1)
        def _():
            count_with(oidx)

        plsc.subcore_barrier()

        @pl.when(cid == 0)
        def _():
            pltpu.sync_copy(acc.at[pl.ds(r0, rpt)], cnt_s.at[pl.ds(r0, rpt)])

        @pl.when(cid == 1)
        def _():
            pltpu.sync_copy(acc.at[pl.ds(r0, rpt)], cnt_o.at[pl.ds(r0, rpt)])

    return k


def _sc_counts(s_pad, o_pad, O_pad):
    return _counts_call(s_pad.shape[0], O_pad)(s_pad, o_pad)


@functools.cache
def _scatter_call(T_pad, O_pad, O):
    C = 128
    TPC = T_pad // NTILE
    nch = TPC // C
    assert nch % 2 == 0
    R_ACC = _round_up(O + 1, 128)
    rpt = R_ACC // NTILE

    @functools.partial(
        pl.kernel, mesh=_mesh(),
        out_type=jax.ShapeDtypeStruct((O_pad, 4 * D), jnp.float32),
        scratch_types=[pltpu.VMEM((C,), jnp.int32),
                       pltpu.VMEM((C,), jnp.int32),
                       pltpu.VMEM((C, D), jnp.float32),
                       pltpu.VMEM((C, D), jnp.float32),
                       pltpu.VMEM((C, D), jnp.float32),
                       pltpu.VMEM_SHARED((R_ACC, D), jnp.float32),
                       pltpu.SemaphoreType.DMA,
                       pltpu.SemaphoreType.DMA])
    def k(ns, no, sidx, oidx, out,
          is0, io0, rs0, ro0, zbuf, acc,
          sf0, sf1):
        cid = lax.axis_index("c")
        sid = lax.axis_index("s")
        zero16 = jnp.zeros((16,), jnp.float32)

        def zb(r, carry):
            for c8 in range(D // 16):
                zbuf[r, pl.ds(c8 * 16, 16)] = zero16
            return carry

        lax.fori_loop(0, C, zb, 0)
        r0 = sid * rpt
        e_base = sid * TPC

        for cc_local in range(2):
            cc = cid * 2 + cc_local
            col0 = cc * D

            def zacc(j, carry):
                pltpu.sync_copy(zbuf, acc.at[pl.ds(r0 + j * C, C)])
                return carry

            lax.fori_loop(0, rpt // C, zacc, 0)
            rem = rpt - (rpt // C) * C
            if rem:
                pltpu.sync_copy(zbuf.at[pl.ds(0, rem)],
                                acc.at[pl.ds(r0 + rpt - rem, rem)])
            plsc.subcore_barrier()

            def fetch_s(j):
                e0 = e_base + j * C
                pltpu.async_copy(sidx.at[pl.ds(e0, C)], is0, sf0)
                pltpu.async_copy(ns.at[pl.ds(e0, C), pl.ds(col0, D)],
                                 rs0, sf0)

            def wait_s(j):
                e0 = e_base + j * C
                pltpu.make_async_copy(
                    sidx.at[pl.ds(e0, C)], is0, sf0).wait()
                pltpu.make_async_copy(
                    ns.at[pl.ds(e0, C), pl.ds(col0, D)], rs0, sf0).wait()

            def fetch_o(j):
                e0 = e_base + j * C
                pltpu.async_copy(oidx.at[pl.ds(e0, C)], io0, sf1)
                pltpu.async_copy(no.at[pl.ds(e0, C), pl.ds(col0, D)],
                                 ro0, sf1)

            def wait_o(j):
                e0 = e_base + j * C
                pltpu.make_async_copy(
                    oidx.at[pl.ds(e0, C)], io0, sf1).wait()
                pltpu.make_async_copy(
                    no.at[pl.ds(e0, C), pl.ds(col0, D)], ro0, sf1).wait()

            fetch_s(0)
            fetch_o(0)

            def body(j, carry):
                nj = jnp.where(j + 1 >= nch, 0, j + 1)
                wait_s(j)
                pltpu.sync_copy(rs0, acc.at[is0], add=True)
                fetch_s(nj)
                wait_o(j)
                pltpu.sync_copy(ro0, acc.at[io0], add=True)
                fetch_o(nj)
                return carry

            lax.fori_loop(0, nch, body, 0)
            wait_s(0)
            wait_o(0)
            plsc.subcore_barrier()
            pltpu.sync_copy(acc.at[pl.ds(r0, rpt)],
                            out.at[pl.ds(r0, rpt), pl.ds(col0, D)])
            plsc.subcore_barrier()

    return k


def _sc_scatter(ns, no, s_pad, o_pad, O_pad, O):
    return _scatter_call(s_pad.shape[0], O_pad, O)(ns, no, s_pad, o_pad)


@functools.cache
def _graph_local_call(T_pad, O_pad):
    @functools.partial(
        pl.kernel, mesh=_mesh(),
        out_type=(jax.ShapeDtypeStruct((NIMG, 16, D), jnp.float32),
                  jax.ShapeDtypeStruct((NIMG, 16, D), jnp.float32),
                  jax.ShapeDtypeStruct((NIMG, 16, D), jnp.float32)),
        scratch_types=[pltpu.VMEM((16,), jnp.int32),
                       pltpu.VMEM((16,), jnp.int32),
                       pltpu.VMEM((32,), jnp.int32),
                       pltpu.VMEM((32,), jnp.int32),
                       pltpu.VMEM((16, D), jnp.float32),
                       pltpu.VMEM((16, D), jnp.float32),
                       pltpu.VMEM((16, D), jnp.float32),
                       pltpu.VMEM((16,), jnp.int32),
                       pltpu.SemaphoreType.DMA])
    def k(aux_e, aux_c, sidx, oidx, ov, pv, out_s, out_p, out_o,
          ebuf, cbuf, swin, owin, bs, bpv, bo, gidx, sem):
        wid = lax.axis_index("s") * 2 + lax.axis_index("c")
        b = wid & (NIMG - 1)
        pltpu.sync_copy(aux_e.at[b, pl.ds(0, 16)], ebuf)
        pltpu.sync_copy(aux_c.at[b, pl.ds(0, 16)], cbuf)
        evec = ebuf[...]
        cvec = cbuf[...]
        start = evec[0]
        lane = lax.broadcasted_iota(jnp.int32, (16,), 0)
        valid = lane < jnp.minimum(cvec, MAXS)
        a0 = pl.multiple_of((start >> 3) << 3, 8)
        off = start - a0
        pltpu.sync_copy(sidx.at[pl.ds(a0, 32)], swin)
        pltpu.sync_copy(oidx.at[pl.ds(a0, 32)], owin)
        svv = swin[pl.ds(off, 16)]
        ovvv = owin[pl.ds(off, 16)]
        zrow = O_pad - 1
        sv = jnp.where(valid, svv, zrow)
        ovv = jnp.where(valid, ovvv, zrow)
        pidx = jnp.where(valid, evec, T_pad - 1)
        gidx[...] = sv
        pltpu.async_copy(ov.at[gidx], bs, sem).wait()
        gidx[...] = pidx
        pltpu.async_copy(pv.at[gidx], bpv, sem).wait()
        gidx[...] = ovv
        pltpu.async_copy(ov.at[gidx], bo, sem).wait()
        pltpu.sync_copy(bs, out_s.at[b])
        pltpu.sync_copy(bpv, out_p.at[b])
        pltpu.sync_copy(bo, out_o.at[b])

    return k


def _sc_graph_local(aux_e, aux_c, s_pad, o_pad, ov, pv):
    return _graph_local_call(s_pad.shape[0], ov.shape[0])(
        aux_e, aux_c, s_pad, o_pad, ov, pv)


def _edge_mlp(gs, pv, go, w1, b1, w2, b2, T, BE=1024):
    T_pad = gs.shape[0]
    grid = (T_pad // BE,)
    f32 = jnp.float32
    bf16 = jnp.bfloat16
    w1h = w1.astype(bf16)
    w1l = (w1 - w1h.astype(f32)).astype(bf16)
    w2h = w2.astype(bf16)
    w2l = (w2 - w2h.astype(f32)).astype(bf16)

    def dot3(x, wh, wl):
        xh = x.astype(bf16)
        xl = (x - xh.astype(f32)).astype(bf16)
        r = jnp.dot(xh, wh, preferred_element_type=f32)
        r += jnp.dot(xl, wh, preferred_element_type=f32)
        r += jnp.dot(xh, wl, preferred_element_type=f32)
        return r

    def kfn(gs_ref, pv_ref, go_ref, w1h_ref, w1l_ref, b1_ref,
            w2h_ref, w2l_ref, b2_ref, ns_ref, npv_ref, no_ref):
        i = pl.program_id(0)
        t = jnp.concatenate([gs_ref[...], pv_ref[...], go_ref[...]], axis=1)
        h = jnp.maximum(dot3(t, w1h_ref[...], w1l_ref[...]) + b1_ref[...],
                        0.0)
        nt = jnp.maximum(dot3(h, w2h_ref[...], w2l_ref[...]) + b2_ref[...],
                         0.0)
        rid = i * BE + lax.broadcasted_iota(jnp.int32, (BE, 1), 0)
        nt = jnp.where(rid < T, nt, 0.0)
        ns_ref[...] = nt[:, :H]
        npv_ref[...] = nt[:, H:H + D]
        no_ref[...] = nt[:, H + D:]

    return pl.pallas_call(
        kfn, grid=grid,
        in_specs=[pl.BlockSpec((BE, D), lambda i: (i, 0)),
                  pl.BlockSpec((BE, D), lambda i: (i, 0)),
                  pl.BlockSpec((BE, D), lambda i: (i, 0)),
                  pl.BlockSpec((3 * D, H), lambda i: (0, 0)),
                  pl.BlockSpec((3 * D, H), lambda i: (0, 0)),
                  pl.BlockSpec((1, H), lambda i: (0, 0)),
                  pl.BlockSpec((H, 2 * H + D), lambda i: (0, 0)),
                  pl.BlockSpec((H, 2 * H + D), lambda i: (0, 0)),
                  pl.BlockSpec((1, 2 * H + D), lambda i: (0, 0))],
        out_specs=[pl.BlockSpec((BE, H), lambda i: (i, 0)),
                   pl.BlockSpec((BE, D), lambda i: (i, 0)),
                   pl.BlockSpec((BE, H), lambda i: (i, 0))],
        out_shape=[jax.ShapeDtypeStruct((T_pad, H), jnp.float32),
                   jax.ShapeDtypeStruct((T_pad, D), jnp.float32),
                   jax.ShapeDtypeStruct((T_pad, H), jnp.float32)],
    )(gs, pv, go, w1h, w1l, b1, w2h, w2l, b2)


def _node_mlp(pooled, cnt_s, cnt_o, w3, b3, w4, b4, O, BO=1024):
    CW_CNT = cnt_s.shape[1]
    O_pad = pooled.shape[0]
    grid = (O_pad // BO,)

    def kfn(pooled_ref, cs_ref, co_ref, w3_ref, b3_ref, w4_ref, b4_ref,
            nov_ref):
        i = pl.program_id(0)
        cnt = cs_ref[...][:, :1] + co_ref[...][:, :1]
        x = pooled_ref[...] * (1.0 / jnp.clip(cnt, 1.0, None))
        h2 = jnp.maximum(
            jnp.dot(x, w3_ref[...], preferred_element_type=jnp.float32,
                    precision=jax.lax.Precision.HIGHEST)
            + b3_ref[...], 0.0)
        nov = jnp.maximum(
            jnp.dot(h2, w4_ref[...], preferred_element_type=jnp.float32,
                    precision=jax.lax.Precision.HIGHEST)
            + b4_ref[...], 0.0)
        rid = i * BO + lax.broadcasted_iota(jnp.int32, (BO, 1), 0)
        nov_ref[...] = jnp.where(rid < O, nov, 0.0)

    return pl.pallas_call(
        kfn, grid=grid,
        in_specs=[pl.BlockSpec((BO, 4 * D), lambda i: (i, 0)),
                  pl.BlockSpec((BO, CW_CNT), lambda i: (i, 0)),
                  pl.BlockSpec((BO, CW_CNT), lambda i: (i, 0)),
                  pl.BlockSpec((H, H), lambda i: (0, 0)),
                  pl.BlockSpec((1, H), lambda i: (0, 0)),
                  pl.BlockSpec((H, D), lambda i: (0, 0)),
                  pl.BlockSpec((1, D), lambda i: (0, 0))],
        out_specs=pl.BlockSpec((BO, D), lambda i: (i, 0)),
        out_shape=jax.ShapeDtypeStruct((O_pad, D), jnp.float32),
    )(pooled, cnt_s, cnt_o, w3, b3, w4, b4)


def _segsum(x, ids3d, BS=1024):
    M = x.shape[0]
    NB = M // BS

    def kfn(x_ref, ids_ref, sum_ref, cnt_ref):
        i = pl.program_id(0)
        ids = ids_ref[0, 0, :]
        oh = (ids[None, :] == lax.broadcasted_iota(
            jnp.int32, (NIMG, BS), 0)).astype(jnp.float32)
        psum = jnp.dot(oh, x_ref[...], preferred_element_type=jnp.float32,
                    precision=jax.lax.Precision.HIGHEST)
        pcnt = jnp.sum(oh, axis=1, keepdims=True)

        @pl.when(i == 0)
        def _():
            sum_ref[...] = jnp.zeros_like(sum_ref)
            cnt_ref[...] = jnp.zeros_like(cnt_ref)

        sum_ref[...] += psum
        cnt_ref[...] += jnp.broadcast_to(pcnt, (NIMG, D))

    return pl.pallas_call(
        kfn, grid=(NB,),
        in_specs=[pl.BlockSpec((BS, D), lambda i: (i, 0)),
                  pl.BlockSpec((1, 1, BS), lambda i: (i, 0, 0))],
        out_specs=[pl.BlockSpec((NIMG, D), lambda i: (0, 0)),
                   pl.BlockSpec((NIMG, D), lambda i: (0, 0))],
        out_shape=[jax.ShapeDtypeStruct((NIMG, D), jnp.float32),
                   jax.ShapeDtypeStruct((NIMG, D), jnp.float32)],
    )(x, ids3d)


def _final(osum, ocnt, psum, pcnt, wp, bp):
    def kfn(os_ref, oc_ref, ps_ref, pc_ref, wp_ref, bp_ref,
            gg_ref, auxe_ref, auxc_ref):
        ofea = os_ref[...] / jnp.clip(oc_ref[...], 1.0, None)
        pfea = ps_ref[...] / jnp.clip(pc_ref[...], 1.0, None)
        cat = jnp.concatenate([ofea, pfea], axis=1)
        gg_ref[...] = (jnp.dot(cat, wp_ref[...],
                               preferred_element_type=jnp.float32,
                    precision=jax.lax.Precision.HIGHEST)
                       + bp_ref[...])
        counts = pc_ref[...][:, :1]
        tril = (lax.broadcasted_iota(jnp.int32, (NIMG, NIMG), 0)
                > lax.broadcasted_iota(jnp.int32, (NIMG, NIMG), 1)
                ).astype(jnp.float32)
        starts = jnp.dot(tril, counts, preferred_element_type=jnp.float32,
                    precision=jax.lax.Precision.HIGHEST)
        lane = lax.broadcasted_iota(jnp.int32, (NIMG, D), 1)
        auxe_ref[...] = starts.astype(jnp.int32) + lane
        auxc_ref[...] = jnp.broadcast_to(counts.astype(jnp.int32), (NIMG, D))

    return pl.pallas_call(
        kfn, grid=(1,),
        in_specs=[pl.BlockSpec((NIMG, D), lambda i: (0, 0)),
                  pl.BlockSpec((NIMG, D), lambda i: (0, 0)),
                  pl.BlockSpec((NIMG, D), lambda i: (0, 0)),
                  pl.BlockSpec((NIMG, D), lambda i: (0, 0)),
                  pl.BlockSpec((2 * D, D), lambda i: (0, 0)),
                  pl.BlockSpec((1, D), lambda i: (0, 0))],
        out_specs=[pl.BlockSpec((NIMG, D), lambda i: (0, 0)),
                   pl.BlockSpec((NIMG, D), lambda i: (0, 0)),
                   pl.BlockSpec((NIMG, D), lambda i: (0, 0))],
        out_shape=[jax.ShapeDtypeStruct((NIMG, D), jnp.float32),
                   jax.ShapeDtypeStruct((NIMG, D), jnp.int32),
                   jax.ShapeDtypeStruct((NIMG, D), jnp.int32)],
    )(osum, ocnt, psum, pcnt, wp, bp)


def kernel(image, objs, boxes, triples, obj_to_img, triples_to_img, params):
    O = objs.shape[0]
    T = triples.shape[0]
    O_pad = _round_up(O, 2048)
    T_pad = _round_up(T + 24, 4096)
    padT = T_pad - T
    i32 = jnp.int32

    s = triples[:, 0].astype(i32)
    p = triples[:, 1].astype(i32)
    o = triples[:, 2].astype(i32)
    s_pad = jnp.concatenate([s, jnp.full((padT,), O, i32)])
    o_pad = jnp.concatenate([o, jnp.full((padT,), O, i32)])
    p_pad = jnp.concatenate([p, jnp.zeros((padT,), i32)])
    objs_pad = jnp.concatenate(
        [objs.astype(i32), jnp.zeros((O_pad - O,), i32)])
    oim3 = jnp.concatenate(
        [obj_to_img.astype(i32), jnp.full((O_pad - O,), NIMG, i32)]
    ).reshape(O_pad // 1024, 1, 1024)
    tim3 = jnp.concatenate(
        [triples_to_img.astype(i32), jnp.full((padT,), NIMG, i32)]
    ).reshape(T_pad // 1024, 1, 1024)

    cnt_s, cnt_o = _sc_counts(s_pad, o_pad, O_pad)

    ov = _sc_gather(params['obj_emb'], objs_pad, 32)
    pv = _sc_gather(params['pred_emb'], p_pad, 128)

    for gp in [params['gconv']] + list(params['gnet']):
        gs = _sc_gather(ov, s_pad, 128)
        go = _sc_gather(ov, o_pad, 128)
        ns, npv, no = _edge_mlp(
            gs, pv, go, gp['W1'], gp['b1'].reshape(1, H),
            gp['W2'], gp['b2'].reshape(1, 2 * H + D), T)
        pooled = _sc_scatter(ns, no, s_pad, o_pad, O_pad, O)
        ov = _node_mlp(pooled, cnt_s, cnt_o, gp['W3'],
                       gp['b3'].reshape(1, H), gp['W4'],
                       gp['b4'].reshape(1, D), O)
        pv = npv

    osum, ocnt = _segsum(ov, oim3)
    psum, pcnt = _segsum(pv, tim3)
    wp, bp = params['proj']
    gg, aux_e, aux_c = _final(osum, ocnt, psum, pcnt, wp, bp.reshape(1, D))
    gl_s, gl_p, gl_o = _sc_graph_local(aux_e, aux_c, s_pad, o_pad, ov, pv)
    glf = jnp.stack([gl_s, gl_p, gl_o], axis=2)
    graph_local = glf[:, :MAXS].reshape(NIMG, MAXS, 3 * D)
    return graph_local, gg


# speedup vs baseline: 2.4445x; 1.0345x over previous
import functools

import jax
import jax.numpy as jnp
from jax import lax
from jax.experimental import pallas as pl
from jax.experimental.pallas import tpu as pltpu
from jax.experimental.pallas import tpu_sc as plsc

D = 128
H = 512
NIMG = 16
MAXS = 15
NW = 32
NTILE = 16


def _round_up(x, m):
    return (x + m - 1) // m * m


def _mesh():
    return plsc.VectorSubcoreMesh(core_axis_name="c", subcore_axis_name="s")


@functools.cache
def _gather_call(N, M, C):
    R = M // NW
    nch = R // C
    assert nch % 2 == 0

    @functools.partial(
        pl.kernel, mesh=_mesh(),
        out_type=jax.ShapeDtypeStruct((M, D), jnp.float32),
        scratch_types=[pltpu.VMEM((C,), jnp.int32),
                       pltpu.VMEM((C,), jnp.int32),
                       pltpu.VMEM((C, D), jnp.float32),
                       pltpu.VMEM((C, D), jnp.float32),
                       pltpu.SemaphoreType.DMA,
                       pltpu.SemaphoreType.DMA,
                       pltpu.SemaphoreType.DMA,
                       pltpu.SemaphoreType.DMA,
                       pltpu.SemaphoreType.DMA,
                       pltpu.SemaphoreType.DMA])
    def k(table, idx, out, i0, i1, r0, r1, si0, si1, sg0, sg1, so0, so1):
        wid = lax.axis_index("s") * 2 + lax.axis_index("c")
        base = wid * R
        ivs, rvs = (i0, i1), (r0, r1)
        sis, sgs, sos = (si0, si1), (sg0, sg1), (so0, so1)

        def idx_cp(j, b):
            return pltpu.make_async_copy(
                idx.at[pl.ds(base + j * C, C)], ivs[b], sis[b])

        def out_cp(j, b):
            return pltpu.make_async_copy(
                rvs[b], out.at[pl.ds(base + j * C, C)], sos[b])

        idx_cp(0, 0).start()
        idx_cp(1, 1).start()

        def body(j2, carry):
            j0 = j2 * 2

            @pl.when(j2 > 0)
            def _():
                out_cp(j0 - 2, 0).wait()
                out_cp(j0 - 1, 1).wait()

            idx_cp(j0, 0).wait()
            g0 = pltpu.make_async_copy(table.at[ivs[0]], rvs[0], sgs[0])
            g0.start()
            idx_cp(j0 + 1, 1).wait()
            g1 = pltpu.make_async_copy(table.at[ivs[1]], rvs[1], sgs[1])
            g1.start()
            g0.wait()
            out_cp(j0, 0).start()
            g1.wait()
            out_cp(j0 + 1, 1).start()
            nj = j0 + 2
            njw = jnp.where(nj >= nch, 0, nj)
            idx_cp(njw, 0).start()
            idx_cp(njw + 1, 1).start()
            return carry

        lax.fori_loop(0, nch // 2, body, 0)
        out_cp(nch - 2, 0).wait()
        out_cp(nch - 1, 1).wait()
        idx_cp(0, 0).wait()
        idx_cp(1, 1).wait()

    return k


def _sc_gather(table, idx, chunk):
    return _gather_call(table.shape[0], idx.shape[0], chunk)(table, idx)


@functools.cache
def _counts_call(T_pad, O_pad):
    C = 128
    TPC = T_pad // NTILE
    nch = TPC // C
    CW = 128
    rpt = O_pad // NTILE

    @functools.partial(
        pl.kernel, mesh=_mesh(),
        out_type=(jax.ShapeDtypeStruct((O_pad, CW), jnp.float32),
                  jax.ShapeDtypeStruct((O_pad, CW), jnp.float32)),
        scratch_types=[pltpu.VMEM((C,), jnp.int32),
                       pltpu.VMEM((C, CW), jnp.float32),
                       pltpu.VMEM((C, CW), jnp.float32),
                       pltpu.VMEM_SHARED((O_pad, CW), jnp.float32),
                       pltpu.SemaphoreType.DMA])
    def k(sidx, oidx, cnt_s, cnt_o, idx_v, ones_v, zbuf, acc, sem):
        cid = lax.axis_index("c")
        sid = lax.axis_index("s")
        one16 = jnp.ones((CW,), jnp.float32)
        zero16 = jnp.zeros((CW,), jnp.float32)

        def fill(r, carry):
            ones_v[r, :] = one16
            zbuf[r, :] = zero16
            return carry

        lax.fori_loop(0, C, fill, 0)

        r0 = sid * rpt

        def zacc(j, carry):
            pltpu.sync_copy(zbuf, acc.at[pl.ds(r0 + j * C, C)])
            return carry

        lax.fori_loop(0, rpt // C, zacc, 0)
        plsc.subcore_barrier()

        def count_with(idxref):
            def body(j, carry):
                e0 = sid * TPC + j * C
                pltpu.sync_copy(idxref.at[pl.ds(e0, C)], idx_v)
                pltpu.sync_copy(ones_v, acc.at[idx_v], add=True)
                return carry
            lax.fori_loop(0, nch, body, 0)

        @pl.when(cid == 0)
        def _():
            count_with(sidx)

        @pl.when(cid == 1)
        def _():
            count_with(oidx)

        plsc.subcore_barrier()

        @pl.when(cid == 0)
        def _():
            pltpu.sync_copy(acc.at[pl.ds(r0, rpt)], cnt_s.at[pl.ds(r0, rpt)])

        @pl.when(cid == 1)
        def _():
            pltpu.sync_copy(acc.at[pl.ds(r0, rpt)], cnt_o.at[pl.ds(r0, rpt)])

    return k


def _sc_counts(s_pad, o_pad, O_pad):
    return _counts_call(s_pad.shape[0], O_pad)(s_pad, o_pad)


@functools.cache
def _scatter_call(T_pad, O_pad, O):
    C = 128
    TPC = T_pad // NTILE
    nch = TPC // C
    assert nch % 2 == 0
    R_ACC = _round_up(O + 1, 128)
    rpt = R_ACC // NTILE

    @functools.partial(
        pl.kernel, mesh=_mesh(),
        out_type=jax.ShapeDtypeStruct((O_pad, 4 * D), jnp.float32),
        scratch_types=[pltpu.VMEM((C,), jnp.int32),
                       pltpu.VMEM((C,), jnp.int32),
                       pltpu.VMEM((C, D), jnp.float32),
                       pltpu.VMEM((C, D), jnp.float32),
                       pltpu.VMEM((C, D), jnp.float32),
                       pltpu.VMEM_SHARED((R_ACC, D), jnp.float32),
                       pltpu.SemaphoreType.DMA,
                       pltpu.SemaphoreType.DMA])
    def k(ns, no, sidx, oidx, out,
          is0, io0, rs0, ro0, zbuf, acc,
          sf0, sf1):
        cid = lax.axis_index("c")
        sid = lax.axis_index("s")
        zero16 = jnp.zeros((16,), jnp.float32)

        def zb(r, carry):
            for c8 in range(D // 16):
                zbuf[r, pl.ds(c8 * 16, 16)] = zero16
            return carry

        lax.fori_loop(0, C, zb, 0)
        r0 = sid * rpt
        e_base = sid * TPC

        for cc_local in range(2):
            cc = cid * 2 + cc_local
            col0 = cc * D

            def zacc(j, carry):
                pltpu.sync_copy(zbuf, acc.at[pl.ds(r0 + j * C, C)])
                return carry

            lax.fori_loop(0, rpt // C, zacc, 0)
            rem = rpt - (rpt // C) * C
            if rem:
                pltpu.sync_copy(zbuf.at[pl.ds(0, rem)],
                                acc.at[pl.ds(r0 + rpt - rem, rem)])
            plsc.subcore_barrier()

            def fetch_s(j):
                e0 = e_base + j * C
                pltpu.async_copy(sidx.at[pl.ds(e0, C)], is0, sf0)
                pltpu.async_copy(ns.at[pl.ds(e0, C), pl.ds(col0, D)],
                                 rs0, sf0)

            def wait_s(j):
                e0 = e_base + j * C
                pltpu.make_async_copy(
                    sidx.at[pl.ds(e0, C)], is0, sf0).wait()
                pltpu.make_async_copy(
                    ns.at[pl.ds(e0, C), pl.ds(col0, D)], rs0, sf0).wait()

            def fetch_o(j):
                e0 = e_base + j * C
                pltpu.async_copy(oidx.at[pl.ds(e0, C)], io0, sf1)
                pltpu.async_copy(no.at[pl.ds(e0, C), pl.ds(col0, D)],
                                 ro0, sf1)

            def wait_o(j):
                e0 = e_base + j * C
                pltpu.make_async_copy(
                    oidx.at[pl.ds(e0, C)], io0, sf1).wait()
                pltpu.make_async_copy(
                    no.at[pl.ds(e0, C), pl.ds(col0, D)], ro0, sf1).wait()

            fetch_s(0)
            fetch_o(0)

            def body(j, carry):
                nj = jnp.where(j + 1 >= nch, 0, j + 1)
                wait_s(j)
                pltpu.sync_copy(rs0, acc.at[is0], add=True)
                fetch_s(nj)
                wait_o(j)
                pltpu.sync_copy(ro0, acc.at[io0], add=True)
                fetch_o(nj)
                return carry

            lax.fori_loop(0, nch, body, 0)
            wait_s(0)
            wait_o(0)
            plsc.subcore_barrier()
            pltpu.sync_copy(acc.at[pl.ds(r0, rpt)],
                            out.at[pl.ds(r0, rpt), pl.ds(col0, D)])
            plsc.subcore_barrier()

    return k


def _sc_scatter(ns, no, s_pad, o_pad, O_pad, O):
    return _scatter_call(s_pad.shape[0], O_pad, O)(ns, no, s_pad, o_pad)


@functools.cache
def _graph_local_call(T_pad, O_pad):
    @functools.partial(
        pl.kernel, mesh=_mesh(),
        out_type=(jax.ShapeDtypeStruct((NIMG, 16, D), jnp.float32),
                  jax.ShapeDtypeStruct((NIMG, 16, D), jnp.float32),
                  jax.ShapeDtypeStruct((NIMG, 16, D), jnp.float32)),
        scratch_types=[pltpu.VMEM((16,), jnp.int32),
                       pltpu.VMEM((16,), jnp.int32),
                       pltpu.VMEM((32,), jnp.int32),
                       pltpu.VMEM((32,), jnp.int32),
                       pltpu.VMEM((16, D), jnp.float32),
                       pltpu.VMEM((16, D), jnp.float32),
                       pltpu.VMEM((16, D), jnp.float32),
                       pltpu.VMEM((16,), jnp.int32),
                       pltpu.SemaphoreType.DMA])
    def k(aux_e, aux_c, sidx, oidx, ov, pv, out_s, out_p, out_o,
          ebuf, cbuf, swin, owin, bs, bpv, bo, gidx, sem):
        wid = lax.axis_index("s") * 2 + lax.axis_index("c")
        b = wid & (NIMG - 1)
        pltpu.sync_copy(aux_e.at[b, pl.ds(0, 16)], ebuf)
        pltpu.sync_copy(aux_c.at[b, pl.ds(0, 16)], cbuf)
        evec = ebuf[...]
        cvec = cbuf[...]
        start = evec[0]
        lane = lax.broadcasted_iota(jnp.int32, (16,), 0)
        valid = lane < jnp.minimum(cvec, MAXS)
        a0 = pl.multiple_of((start >> 3) << 3, 8)
        off = start - a0
        pltpu.sync_copy(sidx.at[pl.ds(a0, 32)], swin)
        pltpu.sync_copy(oidx.at[pl.ds(a0, 32)], owin)
        svv = swin[pl.ds(off, 16)]
        ovvv = owin[pl.ds(off, 16)]
        zrow = O_pad - 1
        sv = jnp.where(valid, svv, zrow)
        ovv = jnp.where(valid, ovvv, zrow)
        pidx = jnp.where(valid, evec, T_pad - 1)
        gidx[...] = sv
        pltpu.async_copy(ov.at[gidx], bs, sem).wait()
        gidx[...] = pidx
        pltpu.async_copy(pv.at[gidx], bpv, sem).wait()
        gidx[...] = ovv
        pltpu.async_copy(ov.at[gidx], bo, sem).wait()
        pltpu.sync_copy(bs, out_s.at[b])
        pltpu.sync_copy(bpv, out_p.at[b])
        pltpu.sync_copy(bo, out_o.at[b])

    return k


def _sc_graph_local(aux_e, aux_c, s_pad, o_pad, ov, pv):
    return _graph_local_call(s_pad.shape[0], ov.shape[0])(
        aux_e, aux_c, s_pad, o_pad, ov, pv)


def _edge_mlp(gs, pv, go, w1, b1, w2, b2, T, BE=1024):
    T_pad = gs.shape[0]
    grid = (T_pad // BE,)
    f32 = jnp.float32
    bf16 = jnp.bfloat16
    w1h = w1.astype(bf16)
    w1l = (w1 - w1h.astype(f32)).astype(bf16)
    w2h = w2.astype(bf16)
    w2l = (w2 - w2h.astype(f32)).astype(bf16)

    def dot3(x, wh, wl):
        xh = x.astype(bf16)
        xl = (x - xh.astype(f32)).astype(bf16)
        r = jnp.dot(xh, wh, preferred_element_type=f32)
        r += jnp.dot(xl, wh, preferred_element_type=f32)
        r += jnp.dot(xh, wl, preferred_element_type=f32)
        return r

    def kfn(gs_ref, pv_ref, go_ref, w1h_ref, w1l_ref, b1_ref,
            w2h_ref, w2l_ref, b2_ref, ns_ref, npv_ref, no_ref):
        i = pl.program_id(0)
        t = jnp.concatenate([gs_ref[...], pv_ref[...], go_ref[...]], axis=1)
        h = jnp.maximum(dot3(t, w1h_ref[...], w1l_ref[...]) + b1_ref[...],
                        0.0)
        nt = jnp.maximum(dot3(h, w2h_ref[...], w2l_ref[...]) + b2_ref[...],
                         0.0)
        rid = i * BE + lax.broadcasted_iota(jnp.int32, (BE, 1), 0)
        nt = jnp.where(rid < T, nt, 0.0)
        ns_ref[...] = nt[:, :H]
        npv_ref[...] = nt[:, H:H + D]
        no_ref[...] = nt[:, H + D:]

    return pl.pallas_call(
        kfn, grid=grid,
        in_specs=[pl.BlockSpec((BE, D), lambda i: (i, 0)),
                  pl.BlockSpec((BE, D), lambda i: (i, 0)),
                  pl.BlockSpec((BE, D), lambda i: (i, 0)),
                  pl.BlockSpec((3 * D, H), lambda i: (0, 0)),
                  pl.BlockSpec((3 * D, H), lambda i: (0, 0)),
                  pl.BlockSpec((1, H), lambda i: (0, 0)),
                  pl.BlockSpec((H, 2 * H + D), lambda i: (0, 0)),
                  pl.BlockSpec((H, 2 * H + D), lambda i: (0, 0)),
                  pl.BlockSpec((1, 2 * H + D), lambda i: (0, 0))],
        out_specs=[pl.BlockSpec((BE, H), lambda i: (i, 0)),
                   pl.BlockSpec((BE, D), lambda i: (i, 0)),
                   pl.BlockSpec((BE, H), lambda i: (i, 0))],
        out_shape=[jax.ShapeDtypeStruct((T_pad, H), jnp.float32),
                   jax.ShapeDtypeStruct((T_pad, D), jnp.float32),
                   jax.ShapeDtypeStruct((T_pad, H), jnp.float32)],
    )(gs, pv, go, w1h, w1l, b1, w2h, w2l, b2)


def _node_mlp(pooled, cnt_s, cnt_o, w3, b3, w4, b4, O, BO=1024):
    CW_CNT = cnt_s.shape[1]
    O_pad = pooled.shape[0]
    grid = (O_pad // BO,)
    f32 = jnp.float32
    bf16 = jnp.bfloat16
    w3h = w3.astype(bf16)
    w3l = (w3 - w3h.astype(f32)).astype(bf16)
    w4h = w4.astype(bf16)
    w4l = (w4 - w4h.astype(f32)).astype(bf16)

    def dot3(x, wh, wl):
        xh = x.astype(bf16)
        xl = (x - xh.astype(f32)).astype(bf16)
        r = jnp.dot(xh, wh, preferred_element_type=f32)
        r += jnp.dot(xl, wh, preferred_element_type=f32)
        r += jnp.dot(xh, wl, preferred_element_type=f32)
        return r

    def kfn(pooled_ref, cs_ref, co_ref, w3h_ref, w3l_ref, b3_ref,
            w4h_ref, w4l_ref, b4_ref, nov_ref):
        i = pl.program_id(0)
        cnt = cs_ref[...][:, :1] + co_ref[...][:, :1]
        x = pooled_ref[...] * (1.0 / jnp.clip(cnt, 1.0, None))
        h2 = jnp.maximum(dot3(x, w3h_ref[...], w3l_ref[...]) + b3_ref[...],
                         0.0)
        nov = jnp.maximum(dot3(h2, w4h_ref[...], w4l_ref[...]) + b4_ref[...],
                          0.0)
        rid = i * BO + lax.broadcasted_iota(jnp.int32, (BO, 1), 0)
        nov_ref[...] = jnp.where(rid < O, nov, 0.0)

    return pl.pallas_call(
        kfn, grid=grid,
        in_specs=[pl.BlockSpec((BO, 4 * D), lambda i: (i, 0)),
                  pl.BlockSpec((BO, CW_CNT), lambda i: (i, 0)),
                  pl.BlockSpec((BO, CW_CNT), lambda i: (i, 0)),
                  pl.BlockSpec((H, H), lambda i: (0, 0)),
                  pl.BlockSpec((H, H), lambda i: (0, 0)),
                  pl.BlockSpec((1, H), lambda i: (0, 0)),
                  pl.BlockSpec((H, D), lambda i: (0, 0)),
                  pl.BlockSpec((H, D), lambda i: (0, 0)),
                  pl.BlockSpec((1, D), lambda i: (0, 0))],
        out_specs=pl.BlockSpec((BO, D), lambda i: (i, 0)),
        out_shape=jax.ShapeDtypeStruct((O_pad, D), jnp.float32),
    )(pooled, cnt_s, cnt_o, w3h, w3l, b3, w4h, w4l, b4)


def _segsum(x, ids3d, BS=1024):
    M = x.shape[0]
    NB = M // BS

    def kfn(x_ref, ids_ref, sum_ref, cnt_ref):
        i = pl.program_id(0)
        ids = ids_ref[0, 0, :]
        oh = (ids[None, :] == lax.broadcasted_iota(
            jnp.int32, (NIMG, BS), 0)).astype(jnp.float32)
        psum = jnp.dot(oh, x_ref[...], preferred_element_type=jnp.float32,
                    precision=jax.lax.Precision.HIGHEST)
        pcnt = jnp.sum(oh, axis=1, keepdims=True)

        @pl.when(i == 0)
        def _():
            sum_ref[...] = jnp.zeros_like(sum_ref)
            cnt_ref[...] = jnp.zeros_like(cnt_ref)

        sum_ref[...] += psum
        cnt_ref[...] += jnp.broadcast_to(pcnt, (NIMG, D))

    return pl.pallas_call(
        kfn, grid=(NB,),
        in_specs=[pl.BlockSpec((BS, D), lambda i: (i, 0)),
                  pl.BlockSpec((1, 1, BS), lambda i: (i, 0, 0))],
        out_specs=[pl.BlockSpec((NIMG, D), lambda i: (0, 0)),
                   pl.BlockSpec((NIMG, D), lambda i: (0, 0))],
        out_shape=[jax.ShapeDtypeStruct((NIMG, D), jnp.float32),
                   jax.ShapeDtypeStruct((NIMG, D), jnp.float32)],
    )(x, ids3d)


def _final(osum, ocnt, psum, pcnt, wp, bp):
    def kfn(os_ref, oc_ref, ps_ref, pc_ref, wp_ref, bp_ref,
            gg_ref, auxe_ref, auxc_ref):
        ofea = os_ref[...] / jnp.clip(oc_ref[...], 1.0, None)
        pfea = ps_ref[...] / jnp.clip(pc_ref[...], 1.0, None)
        cat = jnp.concatenate([ofea, pfea], axis=1)
        gg_ref[...] = (jnp.dot(cat, wp_ref[...],
                               preferred_element_type=jnp.float32,
                    precision=jax.lax.Precision.HIGHEST)
                       + bp_ref[...])
        counts = pc_ref[...][:, :1]
        tril = (lax.broadcasted_iota(jnp.int32, (NIMG, NIMG), 0)
                > lax.broadcasted_iota(jnp.int32, (NIMG, NIMG), 1)
                ).astype(jnp.float32)
        starts = jnp.dot(tril, counts, preferred_element_type=jnp.float32,
                    precision=jax.lax.Precision.HIGHEST)
        lane = lax.broadcasted_iota(jnp.int32, (NIMG, D), 1)
        auxe_ref[...] = starts.astype(jnp.int32) + lane
        auxc_ref[...] = jnp.broadcast_to(counts.astype(jnp.int32), (NIMG, D))

    return pl.pallas_call(
        kfn, grid=(1,),
        in_specs=[pl.BlockSpec((NIMG, D), lambda i: (0, 0)),
                  pl.BlockSpec((NIMG, D), lambda i: (0, 0)),
                  pl.BlockSpec((NIMG, D), lambda i: (0, 0)),
                  pl.BlockSpec((NIMG, D), lambda i: (0, 0)),
                  pl.BlockSpec((2 * D, D), lambda i: (0, 0)),
                  pl.BlockSpec((1, D), lambda i: (0, 0))],
        out_specs=[pl.BlockSpec((NIMG, D), lambda i: (0, 0)),
                   pl.BlockSpec((NIMG, D), lambda i: (0, 0)),
                   pl.BlockSpec((NIMG, D), lambda i: (0, 0))],
        out_shape=[jax.ShapeDtypeStruct((NIMG, D), jnp.float32),
                   jax.ShapeDtypeStruct((NIMG, D), jnp.int32),
                   jax.ShapeDtypeStruct((NIMG, D), jnp.int32)],
    )(osum, ocnt, psum, pcnt, wp, bp)


def kernel(image, objs, boxes, triples, obj_to_img, triples_to_img, params):
    O = objs.shape[0]
    T = triples.shape[0]
    O_pad = _round_up(O, 2048)
    T_pad = _round_up(T + 24, 4096)
    padT = T_pad - T
    i32 = jnp.int32

    s = triples[:, 0].astype(i32)
    p = triples[:, 1].astype(i32)
    o = triples[:, 2].astype(i32)
    s_pad = jnp.concatenate([s, jnp.full((padT,), O, i32)])
    o_pad = jnp.concatenate([o, jnp.full((padT,), O, i32)])
    p_pad = jnp.concatenate([p, jnp.zeros((padT,), i32)])
    objs_pad = jnp.concatenate(
        [objs.astype(i32), jnp.zeros((O_pad - O,), i32)])
    oim3 = jnp.concatenate(
        [obj_to_img.astype(i32), jnp.full((O_pad - O,), NIMG, i32)]
    ).reshape(O_pad // 1024, 1, 1024)
    tim3 = jnp.concatenate(
        [triples_to_img.astype(i32), jnp.full((padT,), NIMG, i32)]
    ).reshape(T_pad // 1024, 1, 1024)

    cnt_s, cnt_o = _sc_counts(s_pad, o_pad, O_pad)

    ov = _sc_gather(params['obj_emb'], objs_pad, 32)
    pv = _sc_gather(params['pred_emb'], p_pad, 128)

    for gp in [params['gconv']] + list(params['gnet']):
        gs = _sc_gather(ov, s_pad, 128)
        go = _sc_gather(ov, o_pad, 128)
        ns, npv, no = _edge_mlp(
            gs, pv, go, gp['W1'], gp['b1'].reshape(1, H),
            gp['W2'], gp['b2'].reshape(1, 2 * H + D), T)
        pooled = _sc_scatter(ns, no, s_pad, o_pad, O_pad, O)
        ov = _node_mlp(pooled, cnt_s, cnt_o, gp['W3'],
                       gp['b3'].reshape(1, H), gp['W4'],
                       gp['b4'].reshape(1, D), O)
        pv = npv

    osum, ocnt = _segsum(ov, oim3)
    psum, pcnt = _segsum(pv, tim3)
    wp, bp = params['proj']
    gg, aux_e, aux_c = _final(osum, ocnt, psum, pcnt, wp, bp.reshape(1, D))
    gl_s, gl_p, gl_o = _sc_graph_local(aux_e, aux_c, s_pad, o_pad, ov, pv)
    glf = jnp.stack([gl_s, gl_p, gl_o], axis=2)
    graph_local = glf[:, :MAXS].reshape(NIMG, MAXS, 3 * D)
    return graph_local, gg
